# Initial kernel scaffold; baseline (speedup 1.0000x reference)
import functools

import jax
import jax.numpy as jnp
from jax import lax
from jax.experimental import pallas as pl
from jax.experimental.pallas import tpu as pltpu
from jax.experimental.pallas import tpu_sc as plsc

N_GRID = 10000
N_MESH = 10000
E = 320000
D = 128
H = 128

NC = 2
NS = 16
NW = NC * NS
EPW = E // NW
BG = 200
BS = 200
RPT = N_MESH // NS

F32 = jnp.float32


def _ln(y, g, beta):
    mu = jnp.mean(y, axis=-1, keepdims=True)
    var = jnp.mean((y - mu) ** 2, axis=-1, keepdims=True)
    return g * (y - mu) * lax.rsqrt(var + 1e-5) + beta


def _dot(a, b):
    return jnp.dot(a, b, preferred_element_type=F32)


def _prep_body(grid_ref, mesh_ref, w1b_ref, w1c_ref, sw1_ref, sb1_ref,
               sw2_ref, sb2_ref, sg_ref, sbeta_ref,
               gh_ref, mh_ref, gout_ref):
    g = grid_ref[...]
    m = mesh_ref[...]
    gh_ref[...] = _dot(g, w1b_ref[...])
    mh_ref[...] = _dot(m, w1c_ref[...])
    x = _dot(g, sw1_ref[...]) + sb1_ref[...]
    h = x * jax.nn.sigmoid(x)
    y = _dot(h, sw2_ref[...]) + sb2_ref[...]
    gout_ref[...] = g + _ln(y, sg_ref[...], sbeta_ref[...])


def _prep(grid, mesh, w1b, w1c, sw1, sb1, sw2, sb2, sg, sbeta):
    n = grid.shape[0]
    bn = 1000
    row = lambda i: (i, 0)
    full = lambda i: (0, 0)
    spec_row = pl.BlockSpec((bn, D), row)
    spec_w = pl.BlockSpec((D, H), full)
    spec_v = pl.BlockSpec((1, D), full)
    return pl.pallas_call(
        _prep_body,
        grid=(n // bn,),
        in_specs=[spec_row, spec_row, spec_w, spec_w, spec_w, spec_v,
                  spec_w, spec_v, spec_v, spec_v],
        out_specs=[spec_row, spec_row, spec_row],
        out_shape=[jax.ShapeDtypeStruct((n, H), F32)] * 2
        + [jax.ShapeDtypeStruct((n, D), F32)],
    )(grid, mesh, w1b, w1c, sw1, sb1, sw2, sb2, sg, sbeta)


def _gather_body(gh_hbm, mh_hbm, src_hbm, dst_hbm, gs_hbm, ms_hbm,
                 si_v, di_v, gr_v, mr_v, sem1, sem2):
    wid = lax.axis_index("s") * NC + lax.axis_index("c")
    base = wid * EPW

    def body(i, carry):
        off = base + i * BG
        pltpu.sync_copy(src_hbm.at[pl.ds(off, BG)], si_v)
        pltpu.sync_copy(dst_hbm.at[pl.ds(off, BG)], di_v)
        c1 = pltpu.async_copy(gh_hbm.at[si_v], gr_v, sem1)
        c2 = pltpu.async_copy(mh_hbm.at[di_v], mr_v, sem2)
        c1.wait()
        c2.wait()
        pltpu.sync_copy(gr_v, gs_hbm.at[pl.ds(off, BG)])
        pltpu.sync_copy(mr_v, ms_hbm.at[pl.ds(off, BG)])
        return carry

    lax.fori_loop(0, EPW // BG, body, 0)


_gather = functools.partial(
    pl.kernel,
    _gather_body,
    out_type=[jax.ShapeDtypeStruct((E, H), F32),
              jax.ShapeDtypeStruct((E, H), F32)],
    mesh=plsc.VectorSubcoreMesh(core_axis_name="c", subcore_axis_name="s"),
    scratch_types=[pltpu.VMEM((BG,), jnp.int32),
                   pltpu.VMEM((BG,), jnp.int32),
                   pltpu.VMEM((BG, H), F32),
                   pltpu.VMEM((BG, H), F32),
                   pltpu.SemaphoreType.DMA,
                   pltpu.SemaphoreType.DMA],
)()


def _edge_body(ef_ref, gs_ref, ms_ref, w1a_ref, b1_ref, w2_ref, b2_ref,
               g_ref, beta_ref, out_ref):
    x = _dot(ef_ref[...], w1a_ref[...]) + b1_ref[...] + gs_ref[...] + ms_ref[...]
    h = x * jax.nn.sigmoid(x)
    y = _dot(h, w2_ref[...]) + b2_ref[...]
    out_ref[...] = _ln(y, g_ref[...], beta_ref[...])


def _edge(efeat, gs, ms, w1a, b1, w2, b2, g, beta):
    br = 1280
    row = lambda i: (i, 0)
    full = lambda i: (0, 0)
    spec_row = pl.BlockSpec((br, D), row)
    spec_w = pl.BlockSpec((D, H), full)
    spec_v = pl.BlockSpec((1, D), full)
    return pl.pallas_call(
        _edge_body,
        grid=(E // br,),
        in_specs=[spec_row, spec_row, spec_row, spec_w, spec_v, spec_w,
                  spec_v, spec_v, spec_v],
        out_specs=spec_row,
        out_shape=jax.ShapeDtypeStruct((E, D), F32),
    )(efeat, gs, ms, w1a, b1, w2, b2, g, beta)


def _scatter_body(eout_hbm, dst_hbm, zeros_hbm, part_hbm,
                  di_v, er_v, acc_sh, sem):
    c = lax.axis_index("c")
    s = lax.axis_index("s")
    wid = s * NC + c
    base = wid * EPW
    pltpu.sync_copy(zeros_hbm.at[pl.ds(s * RPT, RPT)],
                    acc_sh.at[pl.ds(s * RPT, RPT)])
    plsc.subcore_barrier()

    def body(i, carry):
        off = base + i * BS
        pltpu.sync_copy(dst_hbm.at[pl.ds(off, BS)], di_v)
        pltpu.sync_copy(eout_hbm.at[pl.ds(off, BS)], er_v)
        pltpu.sync_copy(er_v, acc_sh.at[di_v], add=True)
        return carry

    lax.fori_loop(0, EPW // BS, body, 0)
    plsc.subcore_barrier()
    pltpu.sync_copy(acc_sh.at[pl.ds(s * RPT, RPT)],
                    part_hbm.at[c].at[pl.ds(s * RPT, RPT)])


_scatter = functools.partial(
    pl.kernel,
    _scatter_body,
    out_type=jax.ShapeDtypeStruct((NC, N_MESH, D), F32),
    mesh=plsc.VectorSubcoreMesh(core_axis_name="c", subcore_axis_name="s"),
    scratch_types=[pltpu.VMEM((BS,), jnp.int32),
                   pltpu.VMEM((BS, D), F32),
                   pltpu.VMEM_SHARED((N_MESH, D), F32),
                   pltpu.SemaphoreType.DMA],
)()


def _mesh_body(p_ref, mesh_ref, dw1a_ref, dw1b_ref, db1_ref, dw2_ref,
               db2_ref, dg_ref, dbeta_ref, out_ref):
    agg = p_ref[0] + p_ref[1]
    m = mesh_ref[...]
    x = _dot(agg, dw1a_ref[...]) + _dot(m, dw1b_ref[...]) + db1_ref[...]
    h = x * jax.nn.sigmoid(x)
    y = _dot(h, dw2_ref[...]) + db2_ref[...]
    out_ref[...] = m + _ln(y, dg_ref[...], dbeta_ref[...])


def _mesh(parts, mesh, dw1a, dw1b, db1, dw2, db2, dg, dbeta):
    bn = 1000
    full = lambda i: (0, 0)
    spec_p = pl.BlockSpec((NC, bn, D), lambda i: (0, i, 0))
    spec_row = pl.BlockSpec((bn, D), lambda i: (i, 0))
    spec_w = pl.BlockSpec((D, H), full)
    spec_v = pl.BlockSpec((1, D), full)
    return pl.pallas_call(
        _mesh_body,
        grid=(N_MESH // bn,),
        in_specs=[spec_p, spec_row, spec_w, spec_w, spec_v, spec_w,
                  spec_v, spec_v, spec_v],
        out_specs=spec_row,
        out_shape=jax.ShapeDtypeStruct((N_MESH, D), F32),
    )(parts, mesh, dw1a, dw1b, db1, dw2, db2, dg, dbeta)


def kernel(g2m_efeat, grid_nfeat, mesh_nfeat, edge_index,
           e_W1, e_b1, e_W2, e_b2, e_g, e_beta,
           s_W1, s_b1, s_W2, s_b2, s_g, s_beta,
           d_W1, d_b1, d_W2, d_b2, d_g, d_beta):
    src = edge_index[0].astype(jnp.int32)
    dst = edge_index[1].astype(jnp.int32)
    w1a, w1b, w1c = e_W1[:D], e_W1[D:2 * D], e_W1[2 * D:]
    dw1a, dw1b = d_W1[:D], d_W1[D:]
    r = lambda v: v.reshape(1, -1)

    gh, mh, grid_out = _prep(grid_nfeat, mesh_nfeat, w1b, w1c,
                             s_W1, r(s_b1), s_W2, r(s_b2), r(s_g), r(s_beta))
    gs, ms = _gather(gh, mh, src, dst)
    e_out = _edge(g2m_efeat, gs, ms, w1a, r(e_b1), e_W2, r(e_b2),
                  r(e_g), r(e_beta))
    zeros = jnp.zeros((N_MESH, D), F32)
    parts = _scatter(e_out, dst, zeros)
    mesh_out = _mesh(parts, mesh_nfeat, dw1a, dw1b, r(d_b1), d_W2,
                     r(d_b2), r(d_g), r(d_beta))
    return (grid_out, mesh_out)


# speedup vs baseline: 3.4541x; 3.4541x over previous
import functools

import jax
import jax.numpy as jnp
from jax import lax
from jax.experimental import pallas as pl
from jax.experimental.pallas import tpu as pltpu
from jax.experimental.pallas import tpu_sc as plsc

N_GRID = 10000
N_MESH = 10000
E = 320000
D = 128
H = 128

NC = 2
NS = 16
NW = NC * NS
EPW = E // NW
BG = 200
BS = 200
NP = 10240
RPT = NP // NS

F32 = jnp.float32


def _ln(y, g, beta):
    mu = jnp.mean(y, axis=-1, keepdims=True)
    var = jnp.mean((y - mu) ** 2, axis=-1, keepdims=True)
    return g * (y - mu) * lax.rsqrt(var + 1e-5) + beta


def _dot(a, b):
    return jnp.dot(a, b, preferred_element_type=F32)


def _prep_body(grid_ref, mesh_ref, w1b_ref, w1c_ref, sw1_ref, sb1_ref,
               sw2_ref, sb2_ref, sg_ref, sbeta_ref,
               gh_ref, mh_ref, gout_ref):
    g = grid_ref[...]
    m = mesh_ref[...]
    gh_ref[...] = _dot(g, w1b_ref[...])
    mh_ref[...] = _dot(m, w1c_ref[...])
    x = _dot(g, sw1_ref[...]) + sb1_ref[...]
    h = x * jax.nn.sigmoid(x)
    y = _dot(h, sw2_ref[...]) + sb2_ref[...]
    gout_ref[...] = g + _ln(y, sg_ref[...], sbeta_ref[...])


def _prep(grid, mesh, w1b, w1c, sw1, sb1, sw2, sb2, sg, sbeta):
    n = grid.shape[0]
    bn = 1000
    row = lambda i: (i, 0)
    full = lambda i: (0, 0)
    spec_row = pl.BlockSpec((bn, D), row)
    spec_w = pl.BlockSpec((D, H), full)
    spec_v = pl.BlockSpec((1, D), full)
    return pl.pallas_call(
        _prep_body,
        grid=(n // bn,),
        in_specs=[spec_row, spec_row, spec_w, spec_w, spec_w, spec_v,
                  spec_w, spec_v, spec_v, spec_v],
        out_specs=[spec_row, spec_row, spec_row],
        out_shape=[jax.ShapeDtypeStruct((n, H), F32)] * 2
        + [jax.ShapeDtypeStruct((n, D), F32)],
    )(grid, mesh, w1b, w1c, sw1, sb1, sw2, sb2, sg, sbeta)


def _gather_body(gh_hbm, mh_hbm, src_hbm, dst_hbm, gs_hbm, ms_hbm,
                 si_v, di_v, gr_v, mr_v, sem1, sem2):
    wid = lax.axis_index("s") * NC + lax.axis_index("c")
    base = wid * EPW

    def body(i, carry):
        off = base + i * BG
        pltpu.sync_copy(src_hbm.at[pl.ds(off, BG)], si_v)
        pltpu.sync_copy(dst_hbm.at[pl.ds(off, BG)], di_v)
        c1 = pltpu.async_copy(gh_hbm.at[si_v], gr_v, sem1)
        c2 = pltpu.async_copy(mh_hbm.at[di_v], mr_v, sem2)
        c1.wait()
        c2.wait()
        pltpu.sync_copy(gr_v, gs_hbm.at[pl.ds(off, BG)])
        pltpu.sync_copy(mr_v, ms_hbm.at[pl.ds(off, BG)])
        return carry

    lax.fori_loop(0, EPW // BG, body, 0)


@functools.cache
def _gather_kernel():
    return pl.kernel(
        _gather_body,
        out_type=[jax.ShapeDtypeStruct((E, H), F32),
                  jax.ShapeDtypeStruct((E, H), F32)],
        mesh=plsc.VectorSubcoreMesh(core_axis_name="c", subcore_axis_name="s",
                                    num_cores=NC, num_subcores=NS),
        scratch_types=[pltpu.VMEM((BG,), jnp.int32),
                       pltpu.VMEM((BG,), jnp.int32),
                       pltpu.VMEM((BG, H), F32),
                       pltpu.VMEM((BG, H), F32),
                       pltpu.SemaphoreType.DMA,
                       pltpu.SemaphoreType.DMA],
    )


def _gather(gh, mh, src, dst):
    return _gather_kernel()(gh, mh, src, dst)


def _edge_body(ef_ref, gs_ref, ms_ref, w1a_ref, b1_ref, w2_ref, b2_ref,
               g_ref, beta_ref, out_ref):
    x = _dot(ef_ref[...], w1a_ref[...]) + b1_ref[...] + gs_ref[...] + ms_ref[...]
    h = x * jax.nn.sigmoid(x)
    y = _dot(h, w2_ref[...]) + b2_ref[...]
    out_ref[...] = _ln(y, g_ref[...], beta_ref[...])


def _edge(efeat, gs, ms, w1a, b1, w2, b2, g, beta):
    br = 1280
    row = lambda i: (i, 0)
    full = lambda i: (0, 0)
    spec_row = pl.BlockSpec((br, D), row)
    spec_w = pl.BlockSpec((D, H), full)
    spec_v = pl.BlockSpec((1, D), full)
    return pl.pallas_call(
        _edge_body,
        grid=(E // br,),
        in_specs=[spec_row, spec_row, spec_row, spec_w, spec_v, spec_w,
                  spec_v, spec_v, spec_v],
        out_specs=spec_row,
        out_shape=jax.ShapeDtypeStruct((E, D), F32),
    )(efeat, gs, ms, w1a, b1, w2, b2, g, beta)


def _scatter_body(eout_hbm, dst_hbm, zeros_hbm, part_hbm,
                  di_v, er_v, acc_sh, sem):
    c = lax.axis_index("c")
    s = lax.axis_index("s")
    wid = s * NC + c
    base = wid * EPW
    pltpu.sync_copy(zeros_hbm.at[pl.ds(s * RPT, RPT)],
                    acc_sh.at[pl.ds(s * RPT, RPT)])
    plsc.subcore_barrier()

    def body(i, carry):
        off = base + i * BS
        pltpu.sync_copy(dst_hbm.at[pl.ds(off, BS)], di_v)
        pltpu.sync_copy(eout_hbm.at[pl.ds(off, BS)], er_v)
        pltpu.sync_copy(er_v, acc_sh.at[di_v], add=True)
        return carry

    lax.fori_loop(0, EPW // BS, body, 0)
    plsc.subcore_barrier()
    pltpu.sync_copy(acc_sh.at[pl.ds(s * RPT, RPT)],
                    part_hbm.at[c].at[pl.ds(s * RPT, RPT)])


@functools.cache
def _scatter_kernel():
    return pl.kernel(
        _scatter_body,
        out_type=jax.ShapeDtypeStruct((NC, NP, D), F32),
        mesh=plsc.VectorSubcoreMesh(core_axis_name="c", subcore_axis_name="s",
                                    num_cores=NC, num_subcores=NS),
        scratch_types=[pltpu.VMEM((BS,), jnp.int32),
                       pltpu.VMEM((BS, D), F32),
                       pltpu.VMEM_SHARED((NP, D), F32),
                       pltpu.SemaphoreType.DMA],
    )


def _scatter(e_out, dst, zeros):
    return _scatter_kernel()(e_out, dst, zeros)


def _mesh_body(p_ref, mesh_ref, dw1a_ref, dw1b_ref, db1_ref, dw2_ref,
               db2_ref, dg_ref, dbeta_ref, out_ref):
    agg = p_ref[0] + p_ref[1]
    m = mesh_ref[...]
    x = _dot(agg, dw1a_ref[...]) + _dot(m, dw1b_ref[...]) + db1_ref[...]
    h = x * jax.nn.sigmoid(x)
    y = _dot(h, dw2_ref[...]) + db2_ref[...]
    out_ref[...] = m + _ln(y, dg_ref[...], dbeta_ref[...])


def _mesh(parts, mesh, dw1a, dw1b, db1, dw2, db2, dg, dbeta):
    bn = 1000
    full = lambda i: (0, 0)
    spec_p = pl.BlockSpec((NC, bn, D), lambda i: (0, i, 0))
    spec_row = pl.BlockSpec((bn, D), lambda i: (i, 0))
    spec_w = pl.BlockSpec((D, H), full)
    spec_v = pl.BlockSpec((1, D), full)
    return pl.pallas_call(
        _mesh_body,
        grid=(N_MESH // bn,),
        in_specs=[spec_p, spec_row, spec_w, spec_w, spec_v, spec_w,
                  spec_v, spec_v, spec_v],
        out_specs=spec_row,
        out_shape=jax.ShapeDtypeStruct((N_MESH, D), F32),
    )(parts, mesh, dw1a, dw1b, db1, dw2, db2, dg, dbeta)


def kernel(g2m_efeat, grid_nfeat, mesh_nfeat, edge_index,
           e_W1, e_b1, e_W2, e_b2, e_g, e_beta,
           s_W1, s_b1, s_W2, s_b2, s_g, s_beta,
           d_W1, d_b1, d_W2, d_b2, d_g, d_beta):
    src = edge_index[0].astype(jnp.int32)
    dst = edge_index[1].astype(jnp.int32)
    w1a, w1b, w1c = e_W1[:D], e_W1[D:2 * D], e_W1[2 * D:]
    dw1a, dw1b = d_W1[:D], d_W1[D:]
    r = lambda v: v.reshape(1, -1)

    gh, mh, grid_out = _prep(grid_nfeat, mesh_nfeat, w1b, w1c,
                             s_W1, r(s_b1), s_W2, r(s_b2), r(s_g), r(s_beta))
    gs, ms = _gather(gh, mh, src, dst)
    e_out = _edge(g2m_efeat, gs, ms, w1a, r(e_b1), e_W2, r(e_b2),
                  r(e_g), r(e_beta))
    zeros = jnp.zeros((NP, D), F32)
    parts = _scatter(e_out, dst, zeros)
    mesh_out = _mesh(parts, mesh_nfeat, dw1a, dw1b, r(d_b1), d_W2,
                     r(d_b2), r(d_g), r(d_beta))
    return (grid_out, mesh_out)


# speedup vs baseline: 3.8100x; 1.1030x over previous
import functools

import jax
import jax.numpy as jnp
from jax import lax
from jax.experimental import pallas as pl
from jax.experimental.pallas import tpu as pltpu
from jax.experimental.pallas import tpu_sc as plsc

N_GRID = 10000
N_MESH = 10000
E = 320000
D = 128
H = 128

NC = 2
NS = 16
NW = NC * NS
EPW = E // NW
BG = 80
NBG = EPW // BG
BS = 200
NP = 10240
RPT = NP // NS

F32 = jnp.float32


def _ln(y, g, beta):
    mu = jnp.mean(y, axis=-1, keepdims=True)
    var = jnp.mean((y - mu) ** 2, axis=-1, keepdims=True)
    return g * (y - mu) * lax.rsqrt(var + 1e-5) + beta


def _dot(a, b):
    return jnp.dot(a, b, preferred_element_type=F32)


def _prep_body(grid_ref, mesh_ref, w1b_ref, w1c_ref, sw1_ref, sb1_ref,
               sw2_ref, sb2_ref, sg_ref, sbeta_ref,
               gh_ref, mh_ref, gout_ref):
    g = grid_ref[...]
    m = mesh_ref[...]
    gh_ref[...] = _dot(g, w1b_ref[...])
    mh_ref[...] = _dot(m, w1c_ref[...])
    x = _dot(g, sw1_ref[...]) + sb1_ref[...]
    h = x * jax.nn.sigmoid(x)
    y = _dot(h, sw2_ref[...]) + sb2_ref[...]
    gout_ref[...] = g + _ln(y, sg_ref[...], sbeta_ref[...])


def _prep(grid, mesh, w1b, w1c, sw1, sb1, sw2, sb2, sg, sbeta):
    n = grid.shape[0]
    bn = 1000
    row = lambda i: (i, 0)
    full = lambda i: (0, 0)
    spec_row = pl.BlockSpec((bn, D), row)
    spec_w = pl.BlockSpec((D, H), full)
    spec_v = pl.BlockSpec((1, D), full)
    return pl.pallas_call(
        _prep_body,
        grid=(n // bn,),
        in_specs=[spec_row, spec_row, spec_w, spec_w, spec_w, spec_v,
                  spec_w, spec_v, spec_v, spec_v],
        out_specs=[spec_row, spec_row, spec_row],
        out_shape=[jax.ShapeDtypeStruct((n, H), F32)] * 2
        + [jax.ShapeDtypeStruct((n, D), F32)],
    )(grid, mesh, w1b, w1c, sw1, sb1, sw2, sb2, sg, sbeta)


def _gather_body(gh_hbm, mh_hbm, src_hbm, dst_hbm, s_hbm,
                 si_v, di_v, g_v, m_v, s_v,
                 sg0, sg1, sm0, sm1, sw0, sw1):
    sgs = (sg0, sg1)
    sms = (sm0, sm1)
    sws = (sw0, sw1)
    wid = lax.axis_index("s") * NC + lax.axis_index("c")
    base = wid * EPW

    def start_gather(i, b):
        off = base + i * BG
        pltpu.sync_copy(src_hbm.at[pl.ds(off, BG)], si_v.at[b])
        pltpu.sync_copy(dst_hbm.at[pl.ds(off, BG)], di_v.at[b])
        pltpu.async_copy(gh_hbm.at[si_v.at[b]], g_v.at[b], sgs[b])
        pltpu.async_copy(mh_hbm.at[di_v.at[b]], m_v.at[b], sms[b])

    def wait_gather(b):
        pltpu.make_async_copy(gh_hbm.at[si_v.at[b]], g_v.at[b], sgs[b]).wait()
        pltpu.make_async_copy(mh_hbm.at[di_v.at[b]], m_v.at[b], sms[b]).wait()

    def add_block(b):
        def row(r, carry):
            for c in range(H // 16):
                sl = pl.ds(c * 16, 16)
                s_v[b, r, sl] = g_v[b, r, sl] + m_v[b, r, sl]
            return carry
        lax.fori_loop(0, BG, row, 0)

    def start_write(i, b):
        pltpu.async_copy(s_v.at[b], s_hbm.at[pl.ds(base + i * BG, BG)], sws[b])

    def wait_write(i, b):
        pltpu.make_async_copy(s_v.at[b], s_hbm.at[pl.ds(base + i * BG, BG)],
                              sws[b]).wait()

    start_gather(0, 0)
    start_gather(1, 1)

    def pair(gidx, carry):
        for b in (0, 1):
            i = 2 * gidx + b
            wait_gather(b)

            @pl.when(i >= 2)
            def _():
                wait_write(i - 2, b)

            add_block(b)

            @pl.when(i + 2 < NBG)
            def _():
                start_gather(i + 2, b)

            start_write(i, b)
        return carry

    lax.fori_loop(0, NBG // 2, pair, 0)
    i_last = NBG - 1
    wait_gather(0)
    wait_write(i_last - 2, 0)
    add_block(0)
    start_write(i_last, 0)
    wait_write(i_last - 1, 1)
    wait_write(i_last, 0)


@functools.cache
def _gather_kernel():
    return pl.kernel(
        _gather_body,
        out_type=jax.ShapeDtypeStruct((E, H), F32),
        mesh=plsc.VectorSubcoreMesh(core_axis_name="c", subcore_axis_name="s",
                                    num_cores=NC, num_subcores=NS),
        scratch_types=[pltpu.VMEM((2, BG), jnp.int32),
                       pltpu.VMEM((2, BG), jnp.int32),
                       pltpu.VMEM((2, BG, H), F32),
                       pltpu.VMEM((2, BG, H), F32),
                       pltpu.VMEM((2, BG, H), F32),
                       pltpu.SemaphoreType.DMA,
                       pltpu.SemaphoreType.DMA,
                       pltpu.SemaphoreType.DMA,
                       pltpu.SemaphoreType.DMA,
                       pltpu.SemaphoreType.DMA,
                       pltpu.SemaphoreType.DMA],
    )


def _gather(gh, mh, src, dst):
    return _gather_kernel()(gh, mh, src, dst)


def _edge_body(ef_ref, s_ref, w1a_ref, b1_ref, w2_ref, b2_ref,
               g_ref, beta_ref, out_ref):
    x = _dot(ef_ref[...], w1a_ref[...]) + b1_ref[...] + s_ref[...]
    h = x * jax.nn.sigmoid(x)
    y = _dot(h, w2_ref[...]) + b2_ref[...]
    out_ref[...] = _ln(y, g_ref[...], beta_ref[...])


def _edge(efeat, s, w1a, b1, w2, b2, g, beta):
    br = 1280
    row = lambda i: (i, 0)
    full = lambda i: (0, 0)
    spec_row = pl.BlockSpec((br, D), row)
    spec_w = pl.BlockSpec((D, H), full)
    spec_v = pl.BlockSpec((1, D), full)
    return pl.pallas_call(
        _edge_body,
        grid=(E // br,),
        in_specs=[spec_row, spec_row, spec_w, spec_v, spec_w,
                  spec_v, spec_v, spec_v],
        out_specs=spec_row,
        out_shape=jax.ShapeDtypeStruct((E, D), F32),
    )(efeat, s, w1a, b1, w2, b2, g, beta)


def _scatter_body(eout_hbm, dst_hbm, zeros_hbm, part_hbm,
                  di_v, er_v, acc_sh, sem):
    c = lax.axis_index("c")
    s = lax.axis_index("s")
    wid = s * NC + c
    base = wid * EPW
    pltpu.sync_copy(zeros_hbm.at[pl.ds(s * RPT, RPT)],
                    acc_sh.at[pl.ds(s * RPT, RPT)])
    plsc.subcore_barrier()

    def body(i, carry):
        off = base + i * BS
        pltpu.sync_copy(dst_hbm.at[pl.ds(off, BS)], di_v)
        pltpu.sync_copy(eout_hbm.at[pl.ds(off, BS)], er_v)
        pltpu.sync_copy(er_v, acc_sh.at[di_v], add=True)
        return carry

    lax.fori_loop(0, EPW // BS, body, 0)
    plsc.subcore_barrier()
    pltpu.sync_copy(acc_sh.at[pl.ds(s * RPT, RPT)],
                    part_hbm.at[c].at[pl.ds(s * RPT, RPT)])


@functools.cache
def _scatter_kernel():
    return pl.kernel(
        _scatter_body,
        out_type=jax.ShapeDtypeStruct((NC, NP, D), F32),
        mesh=plsc.VectorSubcoreMesh(core_axis_name="c", subcore_axis_name="s",
                                    num_cores=NC, num_subcores=NS),
        scratch_types=[pltpu.VMEM((BS,), jnp.int32),
                       pltpu.VMEM((BS, D), F32),
                       pltpu.VMEM_SHARED((NP, D), F32),
                       pltpu.SemaphoreType.DMA],
    )


def _scatter(e_out, dst, zeros):
    return _scatter_kernel()(e_out, dst, zeros)


def _mesh_body(p_ref, mesh_ref, dw1a_ref, dw1b_ref, db1_ref, dw2_ref,
               db2_ref, dg_ref, dbeta_ref, out_ref):
    agg = p_ref[0] + p_ref[1]
    m = mesh_ref[...]
    x = _dot(agg, dw1a_ref[...]) + _dot(m, dw1b_ref[...]) + db1_ref[...]
    h = x * jax.nn.sigmoid(x)
    y = _dot(h, dw2_ref[...]) + db2_ref[...]
    out_ref[...] = m + _ln(y, dg_ref[...], dbeta_ref[...])


def _mesh(parts, mesh, dw1a, dw1b, db1, dw2, db2, dg, dbeta):
    bn = 1000
    full = lambda i: (0, 0)
    spec_p = pl.BlockSpec((NC, bn, D), lambda i: (0, i, 0))
    spec_row = pl.BlockSpec((bn, D), lambda i: (i, 0))
    spec_w = pl.BlockSpec((D, H), full)
    spec_v = pl.BlockSpec((1, D), full)
    return pl.pallas_call(
        _mesh_body,
        grid=(N_MESH // bn,),
        in_specs=[spec_p, spec_row, spec_w, spec_w, spec_v, spec_w,
                  spec_v, spec_v, spec_v],
        out_specs=spec_row,
        out_shape=jax.ShapeDtypeStruct((N_MESH, D), F32),
    )(parts, mesh, dw1a, dw1b, db1, dw2, db2, dg, dbeta)


def kernel(g2m_efeat, grid_nfeat, mesh_nfeat, edge_index,
           e_W1, e_b1, e_W2, e_b2, e_g, e_beta,
           s_W1, s_b1, s_W2, s_b2, s_g, s_beta,
           d_W1, d_b1, d_W2, d_b2, d_g, d_beta):
    src = edge_index[0].astype(jnp.int32)
    dst = edge_index[1].astype(jnp.int32)
    w1a, w1b, w1c = e_W1[:D], e_W1[D:2 * D], e_W1[2 * D:]
    dw1a, dw1b = d_W1[:D], d_W1[D:]
    r = lambda v: v.reshape(1, -1)

    gh, mh, grid_out = _prep(grid_nfeat, mesh_nfeat, w1b, w1c,
                             s_W1, r(s_b1), s_W2, r(s_b2), r(s_g), r(s_beta))
    s = _gather(gh, mh, src, dst)
    e_out = _edge(g2m_efeat, s, w1a, r(e_b1),
                  e_W2, r(e_b2), r(e_g), r(e_beta))
    zeros = jnp.zeros((NP, D), F32)
    parts = _scatter(e_out, dst, zeros)
    mesh_out = _mesh(parts, mesh_nfeat, dw1a, dw1b, r(d_b1), d_W2,
                     r(d_b2), r(d_g), r(d_beta))
    return (grid_out, mesh_out)


# speedup vs baseline: 4.4283x; 1.1623x over previous
import functools

import jax
import jax.numpy as jnp
from jax import lax
from jax.experimental import pallas as pl
from jax.experimental.pallas import tpu as pltpu
from jax.experimental.pallas import tpu_sc as plsc

N_GRID = 10000
N_MESH = 10000
E = 320000
D = 128
H = 128

NC = 2
NS = 16
NW = NC * NS
NCHUNK = 2
NE = E // NCHUNK
EPW = NE // NW
BG = 40
NBG = EPW // BG
BS = 200
NP = 10240
RPT = NP // NS

F32 = jnp.float32


def _ln(y, g, beta):
    mu = jnp.mean(y, axis=-1, keepdims=True)
    var = jnp.mean((y - mu) ** 2, axis=-1, keepdims=True)
    return g * (y - mu) * lax.rsqrt(var + 1e-5) + beta


def _dot(a, b):
    return jnp.dot(a, b, preferred_element_type=F32)


def _prep_body(grid_ref, mesh_ref, w1b_ref, w1c_ref, sw1_ref, sb1_ref,
               sw2_ref, sb2_ref, sg_ref, sbeta_ref,
               gh_ref, mh_ref, gout_ref):
    g = grid_ref[...]
    m = mesh_ref[...]
    gh_ref[...] = _dot(g, w1b_ref[...])
    mh_ref[...] = _dot(m, w1c_ref[...])
    x = _dot(g, sw1_ref[...]) + sb1_ref[...]
    h = x * jax.nn.sigmoid(x)
    y = _dot(h, sw2_ref[...]) + sb2_ref[...]
    gout_ref[...] = g + _ln(y, sg_ref[...], sbeta_ref[...])


def _prep(grid, mesh, w1b, w1c, sw1, sb1, sw2, sb2, sg, sbeta):
    n = grid.shape[0]
    bn = 1000
    row = lambda i: (i, 0)
    full = lambda i: (0, 0)
    spec_row = pl.BlockSpec((bn, D), row)
    spec_w = pl.BlockSpec((D, H), full)
    spec_v = pl.BlockSpec((1, D), full)
    return pl.pallas_call(
        _prep_body,
        grid=(n // bn,),
        in_specs=[spec_row, spec_row, spec_w, spec_w, spec_w, spec_v,
                  spec_w, spec_v, spec_v, spec_v],
        out_specs=[spec_row, spec_row, spec_row],
        out_shape=[jax.ShapeDtypeStruct((n, H), F32)] * 2
        + [jax.ShapeDtypeStruct((n, D), F32)],
    )(grid, mesh, w1b, w1c, sw1, sb1, sw2, sb2, sg, sbeta)


def _gather_body(gh_hbm, mh_hbm, src_hbm, dst_hbm, s_hbm,
                 si_v, di_v, g_v, m_v, s_v,
                 sg0, sg1, sm0, sm1, sw0, sw1):
    sgs = (sg0, sg1)
    sms = (sm0, sm1)
    sws = (sw0, sw1)
    wid = lax.axis_index("s") * NC + lax.axis_index("c")
    base = wid * EPW

    def start_gather(i, b):
        off = base + i * BG
        pltpu.sync_copy(src_hbm.at[pl.ds(off, BG)], si_v.at[b])
        pltpu.sync_copy(dst_hbm.at[pl.ds(off, BG)], di_v.at[b])
        pltpu.async_copy(gh_hbm.at[si_v.at[b]], g_v.at[b], sgs[b])
        pltpu.async_copy(mh_hbm.at[di_v.at[b]], m_v.at[b], sms[b])

    def wait_gather(b):
        pltpu.make_async_copy(gh_hbm.at[si_v.at[b]], g_v.at[b], sgs[b]).wait()
        pltpu.make_async_copy(mh_hbm.at[di_v.at[b]], m_v.at[b], sms[b]).wait()

    def add_block(b):
        def row(r, carry):
            for c in range(H // 16):
                sl = pl.ds(c * 16, 16)
                s_v[b, r, sl] = g_v[b, r, sl] + m_v[b, r, sl]
            return carry
        lax.fori_loop(0, BG, row, 0)

    def start_write(i, b):
        pltpu.async_copy(s_v.at[b], s_hbm.at[pl.ds(base + i * BG, BG)], sws[b])

    def wait_write(i, b):
        pltpu.make_async_copy(s_v.at[b], s_hbm.at[pl.ds(base + i * BG, BG)],
                              sws[b]).wait()

    start_gather(0, 0)
    start_gather(1, 1)

    def pair(gidx, carry):
        for b in (0, 1):
            i = 2 * gidx + b
            wait_gather(b)

            @pl.when(i >= 2)
            def _():
                wait_write(i - 2, b)

            add_block(b)

            @pl.when(i + 2 < NBG)
            def _():
                start_gather(i + 2, b)

            start_write(i, b)
        return carry

    lax.fori_loop(0, NBG // 2, pair, 0)
    i_last = NBG - 1
    wait_gather(0)
    wait_write(i_last - 2, 0)
    add_block(0)
    start_write(i_last, 0)
    wait_write(i_last - 1, 1)
    wait_write(i_last, 0)


@functools.cache
def _gather_kernel():
    return pl.kernel(
        _gather_body,
        out_type=jax.ShapeDtypeStruct((NE, H), F32),
        mesh=plsc.VectorSubcoreMesh(core_axis_name="c", subcore_axis_name="s",
                                    num_cores=NC, num_subcores=NS),
        scratch_types=[pltpu.VMEM((2, BG), jnp.int32),
                       pltpu.VMEM((2, BG), jnp.int32),
                       pltpu.VMEM((2, BG, H), F32),
                       pltpu.VMEM((2, BG, H), F32),
                       pltpu.VMEM((2, BG, H), F32),
                       pltpu.SemaphoreType.DMA,
                       pltpu.SemaphoreType.DMA,
                       pltpu.SemaphoreType.DMA,
                       pltpu.SemaphoreType.DMA,
                       pltpu.SemaphoreType.DMA,
                       pltpu.SemaphoreType.DMA],
    )


def _gather(gh, mh, src, dst):
    return _gather_kernel()(gh, mh, src, dst)


def _edge_body(ef_ref, s_ref, w1a_ref, b1_ref, w2_ref, b2_ref,
               g_ref, beta_ref, out_ref):
    x = _dot(ef_ref[...], w1a_ref[...]) + b1_ref[...] + s_ref[...]
    h = x * jax.nn.sigmoid(x)
    y = _dot(h, w2_ref[...]) + b2_ref[...]
    out_ref[...] = _ln(y, g_ref[...], beta_ref[...])


def _edge(efeat, s, w1a, b1, w2, b2, g, beta, chunk):
    br = 1280
    off = chunk * (NE // br)
    full = lambda i: (0, 0)
    spec_ef = pl.BlockSpec((br, D), lambda i: (i + off, 0))
    spec_row = pl.BlockSpec((br, D), lambda i: (i, 0))
    spec_w = pl.BlockSpec((D, H), full)
    spec_v = pl.BlockSpec((1, D), full)
    return pl.pallas_call(
        _edge_body,
        grid=(NE // br,),
        in_specs=[spec_ef, spec_row, spec_w, spec_v, spec_w,
                  spec_v, spec_v, spec_v],
        out_specs=spec_row,
        out_shape=jax.ShapeDtypeStruct((NE, D), F32),
    )(efeat, s, w1a, b1, w2, b2, g, beta)


def _scatter_body(eout_hbm, dst_hbm, init_hbm, part_hbm,
                  di_v, er_v, acc_sh, sem):
    c = lax.axis_index("c")
    s = lax.axis_index("s")
    wid = s * NC + c
    base = wid * EPW
    pltpu.sync_copy(init_hbm.at[c].at[pl.ds(s * RPT, RPT)],
                    acc_sh.at[pl.ds(s * RPT, RPT)])
    plsc.subcore_barrier()

    def body(i, carry):
        off = base + i * BS
        pltpu.sync_copy(dst_hbm.at[pl.ds(off, BS)], di_v)
        pltpu.sync_copy(eout_hbm.at[pl.ds(off, BS)], er_v)
        pltpu.sync_copy(er_v, acc_sh.at[di_v], add=True)
        return carry

    lax.fori_loop(0, EPW // BS, body, 0)
    plsc.subcore_barrier()
    pltpu.sync_copy(acc_sh.at[pl.ds(s * RPT, RPT)],
                    part_hbm.at[c].at[pl.ds(s * RPT, RPT)])


@functools.cache
def _scatter_kernel():
    return pl.kernel(
        _scatter_body,
        out_type=jax.ShapeDtypeStruct((NC, NP, D), F32),
        mesh=plsc.VectorSubcoreMesh(core_axis_name="c", subcore_axis_name="s",
                                    num_cores=NC, num_subcores=NS),
        scratch_types=[pltpu.VMEM((BS,), jnp.int32),
                       pltpu.VMEM((BS, D), F32),
                       pltpu.VMEM_SHARED((NP, D), F32),
                       pltpu.SemaphoreType.DMA],
    )


def _scatter(e_out, dst, init):
    return _scatter_kernel()(e_out, dst, init)


def _mesh_body(p_ref, mesh_ref, dw1a_ref, dw1b_ref, db1_ref, dw2_ref,
               db2_ref, dg_ref, dbeta_ref, out_ref):
    agg = p_ref[0] + p_ref[1]
    m = mesh_ref[...]
    x = _dot(agg, dw1a_ref[...]) + _dot(m, dw1b_ref[...]) + db1_ref[...]
    h = x * jax.nn.sigmoid(x)
    y = _dot(h, dw2_ref[...]) + db2_ref[...]
    out_ref[...] = m + _ln(y, dg_ref[...], dbeta_ref[...])


def _mesh(parts, mesh, dw1a, dw1b, db1, dw2, db2, dg, dbeta):
    bn = 1000
    full = lambda i: (0, 0)
    spec_p = pl.BlockSpec((NC, bn, D), lambda i: (0, i, 0))
    spec_row = pl.BlockSpec((bn, D), lambda i: (i, 0))
    spec_w = pl.BlockSpec((D, H), full)
    spec_v = pl.BlockSpec((1, D), full)
    return pl.pallas_call(
        _mesh_body,
        grid=(N_MESH // bn,),
        in_specs=[spec_p, spec_row, spec_w, spec_w, spec_v, spec_w,
                  spec_v, spec_v, spec_v],
        out_specs=spec_row,
        out_shape=jax.ShapeDtypeStruct((N_MESH, D), F32),
    )(parts, mesh, dw1a, dw1b, db1, dw2, db2, dg, dbeta)


def kernel(g2m_efeat, grid_nfeat, mesh_nfeat, edge_index,
           e_W1, e_b1, e_W2, e_b2, e_g, e_beta,
           s_W1, s_b1, s_W2, s_b2, s_g, s_beta,
           d_W1, d_b1, d_W2, d_b2, d_g, d_beta):
    src = edge_index[0].astype(jnp.int32)
    dst = edge_index[1].astype(jnp.int32)
    w1a, w1b, w1c = e_W1[:D], e_W1[D:2 * D], e_W1[2 * D:]
    dw1a, dw1b = d_W1[:D], d_W1[D:]
    r = lambda v: v.reshape(1, -1)

    gh, mh, grid_out = _prep(grid_nfeat, mesh_nfeat, w1b, w1c,
                             s_W1, r(s_b1), s_W2, r(s_b2), r(s_g), r(s_beta))
    parts = jnp.zeros((NC, NP, D), F32)
    s_chunks = [_gather(gh, mh, src[k * NE:(k + 1) * NE],
                        dst[k * NE:(k + 1) * NE]) for k in range(NCHUNK)]
    for k in range(NCHUNK):
        e_out = _edge(g2m_efeat, s_chunks[k], w1a, r(e_b1), e_W2,
                      r(e_b2), r(e_g), r(e_beta), k)
        parts = _scatter(e_out, dst[k * NE:(k + 1) * NE], parts)
    mesh_out = _mesh(parts, mesh_nfeat, dw1a, dw1b, r(d_b1), d_W2,
                     r(d_b2), r(d_g), r(d_beta))
    return (grid_out, mesh_out)


# speedup vs baseline: 4.9928x; 1.1275x over previous
import functools

import jax
import jax.numpy as jnp
from jax import lax
from jax.experimental import pallas as pl
from jax.experimental.pallas import tpu as pltpu
from jax.experimental.pallas import tpu_sc as plsc

N_GRID = 10000
N_MESH = 10000
E = 320000
D = 128
H = 128

NC = 2
NS = 16
NW = NC * NS
NCHUNK = 2
NE = E // NCHUNK
EPW = NE // NW
BK = 128
NBU = EPW // BK
TAIL = EPW - NBU * BK
NP = 10240
RPT = NP // NS

F32 = jnp.float32


def _ln(y, g, beta):
    mu = jnp.mean(y, axis=-1, keepdims=True)
    var = jnp.mean((y - mu) ** 2, axis=-1, keepdims=True)
    return g * (y - mu) * lax.rsqrt(var + 1e-5) + beta


def _dot(a, b):
    return jnp.dot(a, b, preferred_element_type=F32)


def _prep_body(grid_ref, mesh_ref, w1b_ref, w1c_ref, sw1_ref, sb1_ref,
               sw2_ref, sb2_ref, sg_ref, sbeta_ref,
               gh_ref, mh_ref, gout_ref):
    g = grid_ref[...]
    m = mesh_ref[...]
    gh_ref[...] = _dot(g, w1b_ref[...])
    mh_ref[...] = _dot(m, w1c_ref[...])
    x = _dot(g, sw1_ref[...]) + sb1_ref[...]
    h = x * jax.nn.sigmoid(x)
    y = _dot(h, sw2_ref[...]) + sb2_ref[...]
    gout_ref[...] = g + _ln(y, sg_ref[...], sbeta_ref[...])


def _prep(grid, mesh, w1b, w1c, sw1, sb1, sw2, sb2, sg, sbeta):
    n = grid.shape[0]
    bn = 1000
    row = lambda i: (i, 0)
    full = lambda i: (0, 0)
    spec_row = pl.BlockSpec((bn, D), row)
    spec_w = pl.BlockSpec((D, H), full)
    spec_v = pl.BlockSpec((1, D), full)
    return pl.pallas_call(
        _prep_body,
        grid=(n // bn,),
        in_specs=[spec_row, spec_row, spec_w, spec_w, spec_w, spec_v,
                  spec_w, spec_v, spec_v, spec_v],
        out_specs=[spec_row, spec_row, spec_row],
        out_shape=[jax.ShapeDtypeStruct((n, H), F32)] * 2
        + [jax.ShapeDtypeStruct((n, D), F32)],
    )(grid, mesh, w1b, w1c, sw1, sb1, sw2, sb2, sg, sbeta)


def _gather_body(gh_hbm, mh_hbm, src_hbm, dst_hbm, s_hbm,
                 si_v, di_v, g_v, m_v, s_v,
                 sg0, sg1, sm0, sm1, sw):
    sgs = (sg0, sg1)
    sms = (sm0, sm1)
    wid = lax.axis_index("s") * NC + lax.axis_index("c")
    base = wid * EPW

    def start_gather(i, b):
        off = base + i * BK
        pltpu.sync_copy(src_hbm.at[pl.ds(off, BK)], si_v.at[b])
        pltpu.sync_copy(dst_hbm.at[pl.ds(off, BK)], di_v.at[b])
        pltpu.async_copy(gh_hbm.at[si_v.at[b]], g_v.at[b], sgs[b])
        pltpu.async_copy(mh_hbm.at[di_v.at[b]], m_v.at[b], sms[b])

    def wait_gather(b):
        pltpu.make_async_copy(gh_hbm.at[si_v.at[b]], g_v.at[b], sgs[b]).wait()
        pltpu.make_async_copy(mh_hbm.at[di_v.at[b]], m_v.at[b], sms[b]).wait()

    def add_block(b, nrows):
        def row(r, carry):
            for c in range(H // 16):
                sl = pl.ds(c * 16, 16)
                s_v[r, sl] = g_v[b, r, sl] + m_v[b, r, sl]
            return carry
        lax.fori_loop(0, nrows, row, 0)

    def start_write(i):
        pltpu.async_copy(s_v, s_hbm.at[pl.ds(base + i * BK, BK)], sw)

    def wait_write(i):
        pltpu.make_async_copy(s_v, s_hbm.at[pl.ds(base + i * BK, BK)],
                              sw).wait()

    start_gather(0, 0)
    start_gather(1, 1)

    def pair(gidx, carry):
        for b in (0, 1):
            i = 2 * gidx + b
            wait_gather(b)

            @pl.when(i >= 1)
            def _():
                wait_write(i - 1)

            add_block(b, BK)

            @pl.when(i + 2 < NBU)
            def _():
                start_gather(i + 2, b)

            start_write(i)
        return carry

    lax.fori_loop(0, NBU // 2, pair, 0)
    i_last = NBU - 1
    wait_gather(0)
    wait_write(i_last - 1)
    add_block(0, BK)
    start_write(i_last)
    toff = base + NBU * BK
    pltpu.sync_copy(src_hbm.at[pl.ds(toff, TAIL)], si_v.at[1, pl.ds(0, TAIL)])
    pltpu.sync_copy(dst_hbm.at[pl.ds(toff, TAIL)], di_v.at[1, pl.ds(0, TAIL)])
    c1 = pltpu.async_copy(gh_hbm.at[si_v.at[1, pl.ds(0, TAIL)]],
                          g_v.at[1, pl.ds(0, TAIL)], sgs[1])
    c2 = pltpu.async_copy(mh_hbm.at[di_v.at[1, pl.ds(0, TAIL)]],
                          m_v.at[1, pl.ds(0, TAIL)], sms[1])
    c1.wait()
    c2.wait()
    wait_write(i_last)
    add_block(1, TAIL)
    pltpu.sync_copy(s_v.at[pl.ds(0, TAIL)], s_hbm.at[pl.ds(toff, TAIL)])


@functools.cache
def _gather_kernel():
    return pl.kernel(
        _gather_body,
        out_type=jax.ShapeDtypeStruct((NE, H), F32),
        mesh=plsc.VectorSubcoreMesh(core_axis_name="c", subcore_axis_name="s",
                                    num_cores=NC, num_subcores=NS),
        scratch_types=[pltpu.VMEM((2, BK), jnp.int32),
                       pltpu.VMEM((2, BK), jnp.int32),
                       pltpu.VMEM((2, BK, H), F32),
                       pltpu.VMEM((2, BK, H), F32),
                       pltpu.VMEM((BK, H), F32),
                       pltpu.SemaphoreType.DMA,
                       pltpu.SemaphoreType.DMA,
                       pltpu.SemaphoreType.DMA,
                       pltpu.SemaphoreType.DMA,
                       pltpu.SemaphoreType.DMA],
    )


def _gather(gh, mh, src, dst):
    return _gather_kernel()(gh, mh, src, dst)


def _edge_body(ef_ref, s_ref, w1a_ref, b1_ref, w2_ref, b2_ref,
               g_ref, beta_ref, out_ref):
    x = _dot(ef_ref[...], w1a_ref[...]) + b1_ref[...] + s_ref[...]
    h = x * jax.nn.sigmoid(x)
    y = _dot(h, w2_ref[...]) + b2_ref[...]
    out_ref[...] = _ln(y, g_ref[...], beta_ref[...])


def _edge(efeat, s, w1a, b1, w2, b2, g, beta, chunk):
    br = 1280
    off = chunk * (NE // br)
    full = lambda i: (0, 0)
    spec_ef = pl.BlockSpec((br, D), lambda i: (i + off, 0))
    spec_row = pl.BlockSpec((br, D), lambda i: (i, 0))
    spec_w = pl.BlockSpec((D, H), full)
    spec_v = pl.BlockSpec((1, D), full)
    return pl.pallas_call(
        _edge_body,
        grid=(NE // br,),
        in_specs=[spec_ef, spec_row, spec_w, spec_v, spec_w,
                  spec_v, spec_v, spec_v],
        out_specs=spec_row,
        out_shape=jax.ShapeDtypeStruct((NE, D), F32),
    )(efeat, s, w1a, b1, w2, b2, g, beta)


def _scatter_body(eout_hbm, dst_hbm, zeros_hbm, part_hbm,
                  di_v, er_v, ti_v, acc_sh, sr0, sr1, sa0, sa1):
    srs = (sr0, sr1)
    sas = (sa0, sa1)
    c = lax.axis_index("c")
    s = lax.axis_index("s")
    wid = s * NC + c
    base = wid * EPW
    pltpu.sync_copy(zeros_hbm.at[pl.ds(s * RPT, RPT)],
                    acc_sh.at[pl.ds(s * RPT, RPT)])
    plsc.subcore_barrier()

    def start_read(i, b):
        off = base + i * BK
        pltpu.sync_copy(dst_hbm.at[pl.ds(off, BK)], di_v.at[b])
        pltpu.async_copy(eout_hbm.at[pl.ds(off, BK)], er_v.at[b], srs[b])

    def wait_read(i, b):
        pltpu.make_async_copy(eout_hbm.at[pl.ds(base + i * BK, BK)],
                              er_v.at[b], srs[b]).wait()

    def start_scat(b):
        pltpu.async_copy(er_v.at[b], acc_sh.at[di_v.at[b]], sas[b], add=True)

    def wait_scat(b):
        pltpu.make_async_copy(er_v.at[b], acc_sh.at[di_v.at[b]],
                              sas[b]).wait()

    def pair(gidx, carry):
        for b in (0, 1):
            i = 2 * gidx + b
            b1 = 1 - b

            @pl.when(i >= 2)
            def _():
                wait_scat(b)

            start_read(i, b)

            @pl.when(i >= 1)
            def _():
                wait_read(i - 1, b1)
                start_scat(b1)
        return carry

    lax.fori_loop(0, NBU // 2, pair, 0)
    i_last = NBU - 1
    wait_scat(0)
    start_read(i_last, 0)
    wait_read(i_last - 1, 1)
    start_scat(1)
    wait_read(i_last, 0)
    start_scat(0)
    wait_scat(1)
    toff = base + NBU * BK
    pltpu.sync_copy(dst_hbm.at[pl.ds(toff, TAIL)], ti_v)
    pltpu.sync_copy(eout_hbm.at[pl.ds(toff, TAIL)],
                    er_v.at[1, pl.ds(0, TAIL)])
    wait_scat(0)
    pltpu.sync_copy(er_v.at[1, pl.ds(0, TAIL)],
                    acc_sh.at[ti_v], add=True)

    plsc.subcore_barrier()
    pltpu.sync_copy(acc_sh.at[pl.ds(s * RPT, RPT)],
                    part_hbm.at[c].at[pl.ds(s * RPT, RPT)])


@functools.cache
def _scatter_kernel():
    return pl.kernel(
        _scatter_body,
        out_type=jax.ShapeDtypeStruct((NC, NP, D), F32),
        mesh=plsc.VectorSubcoreMesh(core_axis_name="c", subcore_axis_name="s",
                                    num_cores=NC, num_subcores=NS),
        scratch_types=[pltpu.VMEM((2, BK), jnp.int32),
                       pltpu.VMEM((2, BK, D), F32),
                       pltpu.VMEM((TAIL,), jnp.int32),
                       pltpu.VMEM_SHARED((NP, D), F32),
                       pltpu.SemaphoreType.DMA,
                       pltpu.SemaphoreType.DMA,
                       pltpu.SemaphoreType.DMA,
                       pltpu.SemaphoreType.DMA],
    )


def _scatter(e_out, dst, zeros):
    return _scatter_kernel()(e_out, dst, zeros)


def _mesh_body(p_ref, q_ref, mesh_ref, dw1a_ref, dw1b_ref, db1_ref, dw2_ref,
               db2_ref, dg_ref, dbeta_ref, out_ref):
    agg = (p_ref[0] + p_ref[1]) + (q_ref[0] + q_ref[1])
    m = mesh_ref[...]
    x = _dot(agg, dw1a_ref[...]) + _dot(m, dw1b_ref[...]) + db1_ref[...]
    h = x * jax.nn.sigmoid(x)
    y = _dot(h, dw2_ref[...]) + db2_ref[...]
    out_ref[...] = m + _ln(y, dg_ref[...], dbeta_ref[...])


def _mesh(parts0, parts1, mesh, dw1a, dw1b, db1, dw2, db2, dg, dbeta):
    bn = 1000
    full = lambda i: (0, 0)
    spec_p = pl.BlockSpec((NC, bn, D), lambda i: (0, i, 0))
    spec_row = pl.BlockSpec((bn, D), lambda i: (i, 0))
    spec_w = pl.BlockSpec((D, H), full)
    spec_v = pl.BlockSpec((1, D), full)
    return pl.pallas_call(
        _mesh_body,
        grid=(N_MESH // bn,),
        in_specs=[spec_p, spec_p, spec_row, spec_w, spec_w, spec_v, spec_w,
                  spec_v, spec_v, spec_v],
        out_specs=spec_row,
        out_shape=jax.ShapeDtypeStruct((N_MESH, D), F32),
    )(parts0, parts1, mesh, dw1a, dw1b, db1, dw2, db2, dg, dbeta)


def kernel(g2m_efeat, grid_nfeat, mesh_nfeat, edge_index,
           e_W1, e_b1, e_W2, e_b2, e_g, e_beta,
           s_W1, s_b1, s_W2, s_b2, s_g, s_beta,
           d_W1, d_b1, d_W2, d_b2, d_g, d_beta):
    src = edge_index[0].astype(jnp.int32)
    dst = edge_index[1].astype(jnp.int32)
    w1a, w1b, w1c = e_W1[:D], e_W1[D:2 * D], e_W1[2 * D:]
    dw1a, dw1b = d_W1[:D], d_W1[D:]
    r = lambda v: v.reshape(1, -1)

    gh, mh, grid_out = _prep(grid_nfeat, mesh_nfeat, w1b, w1c,
                             s_W1, r(s_b1), s_W2, r(s_b2), r(s_g), r(s_beta))
    zeros = jnp.zeros((NP, D), F32)
    s_chunks = [_gather(gh, mh, src[k * NE:(k + 1) * NE],
                        dst[k * NE:(k + 1) * NE]) for k in range(NCHUNK)]
    parts = []
    for k in range(NCHUNK):
        e_out = _edge(g2m_efeat, s_chunks[k], w1a, r(e_b1), e_W2,
                      r(e_b2), r(e_g), r(e_beta), k)
        parts.append(_scatter(e_out, dst[k * NE:(k + 1) * NE], zeros))
    mesh_out = _mesh(parts[0], parts[1], mesh_nfeat, dw1a, dw1b, r(d_b1),
                     d_W2, r(d_b2), r(d_g), r(d_beta))
    return (grid_out, mesh_out)
